# baseline (speedup 1.0000x reference)
import functools

import jax
import jax.numpy as jnp
from jax import lax
from jax.experimental import pallas as pl
from jax.experimental.pallas import tpu as pltpu
from jax.experimental.pallas import tpu_sc as plsc

BATCH = 4096
SEQ = 200
EMBED = 64
NUM_TARGETS = 3811

NC, NS = 2, 16
NW = NC * NS
ROWS_PER_W = BATCH // NW
CB = 4
GCH = SEQ // 2
NG = 2 * CB
NBLK = ROWS_PER_W // CB
LANES = 16
CCH = EMBED // LANES


def _pooling_kernel(idx_hbm, table_hbm, out_hbm, idx_v, rows_v, out_v, sem):
    wid = lax.axis_index("s") * NC + lax.axis_index("c")
    base = wid * ROWS_PER_W

    def block(blk, _):
        b0 = base + blk * CB
        pltpu.sync_copy(idx_hbm.at[pl.ds(b0 * 2, NG)], idx_v)
        cps = [
            pltpu.async_copy(table_hbm.at[idx_v.at[j]], rows_v.at[j], sem)
            for j in range(NG)
        ]
        for cp in cps:
            cp.wait()

        def acc_body(r, acc):
            new = []
            for i in range(CB):
                for c in range(CCH):
                    v = acc[i * CCH + c]
                    v = v + rows_v[2 * i, r, pl.ds(LANES * c, LANES)]
                    v = v + rows_v[2 * i + 1, r, pl.ds(LANES * c, LANES)]
                    new.append(v)
            return tuple(new)

        zeros = tuple(
            jnp.zeros((LANES,), jnp.float32) for _ in range(CB * CCH)
        )
        acc = lax.fori_loop(0, GCH, acc_body, zeros)
        scale = jnp.float32(1.0 / SEQ)
        for i in range(CB):
            for c in range(CCH):
                out_v[i, pl.ds(LANES * c, LANES)] = acc[i * CCH + c] * scale
        pltpu.sync_copy(out_v, out_hbm.at[pl.ds(b0, CB)])
        return ()

    lax.fori_loop(0, NBLK, block, ())


@jax.jit
def _pooled_sc(idx2, table):
    mesh = plsc.VectorSubcoreMesh(
        core_axis_name="c", subcore_axis_name="s", num_cores=NC, num_subcores=NS
    )
    return pl.kernel(
        _pooling_kernel,
        out_type=jax.ShapeDtypeStruct((BATCH, EMBED), jnp.float32),
        mesh=mesh,
        compiler_params=pltpu.CompilerParams(use_tc_tiling_on_sc=False),
        scratch_types=[
            pltpu.VMEM((NG, GCH), jnp.int32),
            pltpu.VMEM((NG, GCH, EMBED), jnp.float32),
            pltpu.VMEM((CB, EMBED), jnp.float32),
            pltpu.SemaphoreType.DMA,
        ],
    )(idx2, table)


def _dense_kernel(x_ref, w_ref, b_ref, o_ref):
    y = jnp.dot(x_ref[...], w_ref[...], preferred_element_type=jnp.float32)
    o_ref[...] = jax.nn.sigmoid(y + b_ref[...])


BM = 512
BN = 1280
NPAD = 3840


@jax.jit
def _dense_tc(pooled, wp, bp):
    return pl.pallas_call(
        _dense_kernel,
        grid=(BATCH // BM, NPAD // BN),
        in_specs=[
            pl.BlockSpec((BM, EMBED), lambda i, j: (i, 0)),
            pl.BlockSpec((EMBED, BN), lambda i, j: (0, j)),
            pl.BlockSpec((1, BN), lambda i, j: (0, j)),
        ],
        out_specs=pl.BlockSpec((BM, BN), lambda i, j: (i, j)),
        out_shape=jax.ShapeDtypeStruct((BATCH, NPAD), jnp.float32),
    )(pooled, wp, bp)


def kernel(inputs, table, W, b):
    idx2 = inputs.astype(jnp.int32).reshape(BATCH * 2, GCH)
    pooled = _pooled_sc(idx2, table)
    wp = jnp.pad(W, ((0, 0), (0, NPAD - NUM_TARGETS)))
    bp = jnp.pad(b, (0, NPAD - NUM_TARGETS)).reshape(1, NPAD)
    out = _dense_tc(pooled, wp, bp)
    return out[:, :NUM_TARGETS]
